# Initial kernel scaffold; baseline (speedup 1.0000x reference)
import jax
import jax.numpy as jnp
from jax.experimental import pallas as pl


def kernel(x, Wr, w1, b1, w2, b2):
    raise NotImplementedError("write your pallas kernel here")


# speedup vs baseline: 156.8062x; 156.8062x over previous
import functools

import jax
import jax.numpy as jnp
from jax.experimental import pallas as pl
from jax.experimental.pallas import tpu as pltpu

E = 8
K = 2
CAP_FACTOR = 1.5


def _router_kernel(x_ref, wrt_ref, c_ref, *, capacity):
    x = x_ref[...]
    wrt = wrt_ref[...]
    T = x.shape[0]
    logits = jnp.dot(x, wrt, preferred_element_type=jnp.float32)
    m = jnp.max(logits, axis=-1, keepdims=True)
    ex = jnp.exp(logits - m)
    probs = ex / jnp.sum(ex, axis=-1, keepdims=True)

    lane = jax.lax.broadcasted_iota(jnp.int32, probs.shape, 1)
    p1 = jnp.max(probs, axis=-1, keepdims=True)
    a1 = jnp.min(jnp.where(probs == p1, lane, E), axis=-1, keepdims=True)
    oh1 = (lane == a1).astype(jnp.float32)
    probs2 = jnp.where(lane == a1, -jnp.inf, probs)
    p2 = jnp.max(probs2, axis=-1, keepdims=True)
    a2 = jnp.min(jnp.where(probs2 == p2, lane, E), axis=-1, keepdims=True)
    oh2 = (lane == a2).astype(jnp.float32)

    s = p1 + p2
    p1n = p1 / s
    p2n = p2 / s

    ohsum = oh1 + oh2
    inc = ohsum
    shift = 1
    while shift < T:
        shifted = jnp.concatenate(
            [jnp.zeros((shift, E), jnp.float32), inc[: T - shift]], axis=0)
        inc = inc + shifted
        shift *= 2
    excl = inc - ohsum

    r1 = jnp.sum(oh1 * excl, axis=-1, keepdims=True)
    r2 = jnp.sum(oh2 * excl, axis=-1, keepdims=True)

    keep1 = (r1 < capacity).astype(jnp.float32)
    keep2 = (r2 < capacity).astype(jnp.float32)

    c_ref[...] = oh1 * (p1n * keep1) + oh2 * (p2n * keep2)


def _expert_kernel(x_ref, w1_ref, b1_ref, w2_ref, b2_ref, c_ref, out_ref):
    e = pl.program_id(0)
    x = x_ref[...]
    h = jnp.dot(x, w1_ref[0], preferred_element_type=jnp.float32)
    h = h + b1_ref[0]
    h = 0.5 * h * (1.0 + jax.lax.erf(h * 0.7071067811865476))
    o = jnp.dot(h, w2_ref[0], preferred_element_type=jnp.float32)
    o = o + b2_ref[0]

    lane = jax.lax.broadcasted_iota(jnp.int32, c_ref.shape, 1)
    ce = jnp.sum(jnp.where(lane == e, c_ref[...], 0.0), axis=-1, keepdims=True)
    contrib = o * ce

    @pl.when(e == 0)
    def _():
        out_ref[...] = contrib

    @pl.when(e != 0)
    def _():
        out_ref[...] = out_ref[...] + contrib


@functools.partial(jax.jit, static_argnames=("interpret",))
def _moe(x, Wr, w1, b1, w2, b2, interpret=False):
    B, S, D = x.shape
    T = B * S
    H = w1.shape[-1]
    O = w2.shape[-1]
    capacity = int((T / E) * CAP_FACTOR)

    xt = x.reshape(T, D)
    C = pl.pallas_call(
        functools.partial(_router_kernel, capacity=capacity),
        out_shape=jax.ShapeDtypeStruct((T, E), jnp.float32),
        interpret=interpret,
    )(xt, Wr.T)

    out = pl.pallas_call(
        _expert_kernel,
        grid=(E,),
        in_specs=[
            pl.BlockSpec((T, D), lambda e: (0, 0)),
            pl.BlockSpec((1, D, H), lambda e: (e, 0, 0)),
            pl.BlockSpec((1, 1, H), lambda e: (e, 0, 0)),
            pl.BlockSpec((1, H, O), lambda e: (e, 0, 0)),
            pl.BlockSpec((1, 1, O), lambda e: (e, 0, 0)),
            pl.BlockSpec((T, E), lambda e: (0, 0)),
        ],
        out_specs=pl.BlockSpec((T, O), lambda e: (0, 0)),
        out_shape=jax.ShapeDtypeStruct((T, O), jnp.float32),
        compiler_params=pltpu.CompilerParams(
            dimension_semantics=("arbitrary",),
        ),
        interpret=interpret,
    )(xt, w1, b1.reshape(E, 1, H), w2, b2.reshape(E, 1, O), C)

    return out.reshape(B, S, O)


def kernel(x, Wr, w1, b1, w2, b2):
    return _moe(x, Wr, w1, b1, w2, b2)
